# baseline (speedup 1.0000x reference)
import jax
import jax.numpy as jnp
from jax.experimental import pallas as pl

N = 10000
E = 320000
NG = 128
EPS = 1e-15


def _stack3_kernel(in_ref, o_ref):
    o_ref[...] = in_ref[...]


def kernel(x, edge_index, batch, num_graphs, W0, b0, W1, b1, Wnm, bnm, Wnl, bnl,
           Wcm, bcm, Wcl, bcl, Wd1, bd1, Wd2, bd2):
    src = edge_index[0]
    dst = edge_index[1]
    h = x
    feats = []
    for W, b in ((W0, b0), (W1, b1)):
        agg = jnp.zeros(h.shape, h.dtype).at[dst].add(h[src])
        h = jax.nn.relu((h + agg) @ W + b)
        feats.append(h)
    emb = jnp.concatenate(feats, axis=1)
    node_mu = emb @ Wnm + bnm
    node_logvar = emb @ Wnl + bnl
    class_mu = emb @ Wcm + bcm
    class_logvar = emb @ Wcl + bcl
    node_kl = -0.5 * jnp.sum(1.0 + node_logvar - node_mu ** 2 - jnp.exp(node_logvar))
    node_kl = 1e-07 * node_kl * num_graphs
    cm = jax.lax.stop_gradient(class_mu)
    cl = jax.lax.stop_gradient(class_logvar)
    inv_var = jnp.exp(-cl)
    T = jax.ops.segment_sum(inv_var, batch, num_segments=NG)
    Smu = jax.ops.segment_sum(cm * inv_var, batch, num_segments=NG)
    gvar = 1.0 / (T + 1e-07)
    gmu = gvar * Smu
    glogvar = jnp.log(gvar + EPS)
    class_kl = -0.5 * jnp.sum(1.0 + glogvar - gmu ** 2 - jnp.exp(glogvar))
    class_kl = 1e-07 * class_kl * num_graphs
    kz = jax.random.key(42)
    eps_n = jax.random.normal(jax.random.fold_in(kz, 0), node_mu.shape, node_mu.dtype)
    z_node = node_mu + eps_n * jnp.exp(0.5 * node_logvar)
    eps_g = jax.random.normal(jax.random.fold_in(kz, 1), gmu.shape, gmu.dtype)
    z_class = gmu[batch] + eps_g[batch] * jnp.exp(0.5 * glogvar)[batch]
    h = jnp.concatenate([z_node, z_class], axis=1)
    agg = jnp.zeros(h.shape, h.dtype).at[dst].add(h[src])
    h1 = jax.nn.relu((h + agg) @ Wd1 + bd1)
    recon = h1 @ Wd2 + bd2
    pos = jax.nn.sigmoid(jnp.sum(recon[src] * recon[dst], axis=1))
    pos_loss = -jnp.mean(jnp.log(pos + EPS))
    neg = jax.random.randint(jax.random.fold_in(kz, 2), (2, E), 0, N, dtype=jnp.int32)
    nv = jax.nn.sigmoid(jnp.sum(recon[neg[0]] * recon[neg[1]], axis=1))
    neg_loss = -jnp.mean(jnp.log(1.0 - nv + EPS))
    recon_err = pos_loss + neg_loss

    vals = jnp.stack([recon_err, class_kl, node_kl])
    out = pl.pallas_call(
        _stack3_kernel,
        out_shape=jax.ShapeDtypeStruct((3,), jnp.float32),
    )(vals)
    return out


# speedup vs baseline: 1.8526x; 1.8526x over previous
import functools

import jax
import jax.numpy as jnp
from jax import lax
from jax.experimental import pallas as pl
from jax.experimental.pallas import tpu as pltpu
from jax.experimental.pallas import tpu_sc as plsc

N = 10000
E = 320000
NG = 128
EPS = 1e-15

NC = 2
NS = 16
CHUNK = 80
NP = 10240
ROWS_PER_TILE = NP // NS

_MESH = plsc.VectorSubcoreMesh(
    core_axis_name="c", subcore_axis_name="s", num_cores=NC, num_subcores=NS)


def _sc_agg(table, srcs_flat, dst, width, colsplit):
    ept = E // NS if colsplit else E // (NC * NS)
    nchunk = ept // CHUNK
    zrows = 128

    def body(table_h, srcs_h, dst_h, out_h, acc, idx_s, idx_d, rows, zbuf, sem):
        c = lax.axis_index("c")
        s = lax.axis_index("s")

        zero16 = jnp.zeros((16,), jnp.float32)
        def zrow(i, _):
            for j in range(width // 16):
                zbuf[i, pl.ds(j * 16, 16)] = zero16
            return 0
        lax.fori_loop(0, zrows, zrow, 0)
        row0 = s * ROWS_PER_TILE
        for j in range(ROWS_PER_TILE // zrows):
            pltpu.sync_copy(zbuf, acc.at[pl.ds(row0 + j * zrows, zrows)])
        plsc.subcore_barrier()

        if colsplit:
            src_base = c * E + s * ept
            dst_base = s * ept
        else:
            w = c * NS + s
            src_base = w * ept
            dst_base = w * ept

        def chunk(k, _):
            off = k * CHUNK
            pltpu.sync_copy(srcs_h.at[pl.ds(src_base + off, CHUNK)], idx_s)
            pltpu.async_copy(table_h.at[idx_s], rows, sem).wait()
            pltpu.sync_copy(dst_h.at[pl.ds(dst_base + off, CHUNK)], idx_d)
            pltpu.sync_copy(rows, acc.at[idx_d], add=True)
            return 0
        lax.fori_loop(0, nchunk, chunk, 0)
        plsc.subcore_barrier()

        out_base = c * NP + row0
        for j in range(ROWS_PER_TILE // zrows):
            pltpu.sync_copy(acc.at[pl.ds(row0 + j * zrows, zrows)],
                            out_h.at[pl.ds(out_base + j * zrows, zrows)])

    k = pl.kernel(
        body,
        out_type=jax.ShapeDtypeStruct((NC * NP, width), jnp.float32),
        mesh=_MESH,
        compiler_params=pltpu.CompilerParams(use_tc_tiling_on_sc=False),
        scratch_types=[
            pltpu.VMEM_SHARED((NP, width), jnp.float32),
            pltpu.VMEM((CHUNK,), jnp.int32),
            pltpu.VMEM((CHUNK,), jnp.int32),
            pltpu.VMEM((CHUNK, width), jnp.float32),
            pltpu.VMEM((128, width), jnp.float32),
            pltpu.SemaphoreType.DMA,
        ],
    )
    return k(table, srcs_flat, dst)


def kernel(x, edge_index, batch, num_graphs, W0, b0, W1, b1, Wnm, bnm, Wnl, bnl,
           Wcm, bcm, Wcl, bcl, Wd1, bd1, Wd2, bd2):
    src = edge_index[0]
    dst = edge_index[1]

    p = _sc_agg(x, src, dst, 128, colsplit=False)
    agg1 = p[:N] + p[NP:NP + N]
    h1 = jax.nn.relu((x + agg1) @ W0 + b0)

    p = _sc_agg(h1, src, dst, 64, colsplit=False)
    agg2 = p[:N] + p[NP:NP + N]
    h2 = jax.nn.relu((h1 + agg2) @ W1 + b1)

    emb = jnp.concatenate([h1, h2], axis=1)
    node_mu = emb @ Wnm + bnm
    node_logvar = emb @ Wnl + bnl
    class_mu = emb @ Wcm + bcm
    class_logvar = emb @ Wcl + bcl
    node_kl = -0.5 * jnp.sum(1.0 + node_logvar - node_mu ** 2 - jnp.exp(node_logvar))
    node_kl = 1e-07 * node_kl * num_graphs
    inv_var = jnp.exp(-class_logvar)
    T = jax.ops.segment_sum(inv_var, batch, num_segments=NG)
    Smu = jax.ops.segment_sum(class_mu * inv_var, batch, num_segments=NG)
    gvar = 1.0 / (T + 1e-07)
    gmu = gvar * Smu
    glogvar = jnp.log(gvar + EPS)
    class_kl = -0.5 * jnp.sum(1.0 + glogvar - gmu ** 2 - jnp.exp(glogvar))
    class_kl = 1e-07 * class_kl * num_graphs
    kz = jax.random.key(42)
    eps_n = jax.random.normal(jax.random.fold_in(kz, 0), node_mu.shape, node_mu.dtype)
    z_node = node_mu + eps_n * jnp.exp(0.5 * node_logvar)
    eps_g = jax.random.normal(jax.random.fold_in(kz, 1), gmu.shape, gmu.dtype)
    z_class = gmu[batch] + eps_g[batch] * jnp.exp(0.5 * glogvar)[batch]

    tbl = jnp.concatenate([z_node, z_class], axis=0)
    srcs2 = jnp.concatenate([src, src + N])
    p = _sc_agg(tbl, srcs2, dst, 128, colsplit=True)
    agg3 = jnp.concatenate([p[:N], p[NP:NP + N]], axis=1)

    h = jnp.concatenate([z_node, z_class], axis=1)
    h1d = jax.nn.relu((h + agg3) @ Wd1 + bd1)
    recon = h1d @ Wd2 + bd2
    pos = jax.nn.sigmoid(jnp.sum(recon[src] * recon[dst], axis=1))
    pos_loss = -jnp.mean(jnp.log(pos + EPS))
    neg = jax.random.randint(jax.random.fold_in(kz, 2), (2, E), 0, N, dtype=jnp.int32)
    nv = jax.nn.sigmoid(jnp.sum(recon[neg[0]] * recon[neg[1]], axis=1))
    neg_loss = -jnp.mean(jnp.log(1.0 - nv + EPS))
    recon_err = pos_loss + neg_loss

    return jnp.stack([recon_err, class_kl, node_kl])


# speedup vs baseline: 3.1448x; 1.6975x over previous
import functools

import jax
import jax.numpy as jnp
from jax import lax
from jax.experimental import pallas as pl
from jax.experimental.pallas import tpu as pltpu
from jax.experimental.pallas import tpu_sc as plsc

N = 10000
E = 320000
NG = 128
EPS = 1e-15

NC = 2
NS = 16
CHUNK = 80
NP = 10240
ROWS_PER_TILE = NP // NS

_MESH = plsc.VectorSubcoreMesh(
    core_axis_name="c", subcore_axis_name="s", num_cores=NC, num_subcores=NS)


def _sc_agg(table, srcs_flat, dst, width, colsplit):
    ept = E // NS if colsplit else E // (NC * NS)
    nchunk = ept // CHUNK
    zrows = 128

    def body(table_h, srcs_h, dst_h, out_h, acc, idx_s, idx_d, rows, zbuf, sem):
        c = lax.axis_index("c")
        s = lax.axis_index("s")

        zero16 = jnp.zeros((16,), jnp.float32)
        def zrow(i, _):
            for j in range(width // 16):
                zbuf[i, pl.ds(j * 16, 16)] = zero16
            return 0
        lax.fori_loop(0, zrows, zrow, 0)
        row0 = s * ROWS_PER_TILE
        for j in range(ROWS_PER_TILE // zrows):
            pltpu.sync_copy(zbuf, acc.at[pl.ds(row0 + j * zrows, zrows)])
        plsc.subcore_barrier()

        if colsplit:
            src_base = c * E + s * ept
            dst_base = s * ept
        else:
            w = c * NS + s
            src_base = w * ept
            dst_base = w * ept

        def chunk(k, _):
            off = k * CHUNK
            pltpu.sync_copy(srcs_h.at[pl.ds(src_base + off, CHUNK)], idx_s)
            pltpu.async_copy(table_h.at[idx_s], rows, sem).wait()
            pltpu.sync_copy(dst_h.at[pl.ds(dst_base + off, CHUNK)], idx_d)
            pltpu.sync_copy(rows, acc.at[idx_d], add=True)
            return 0
        lax.fori_loop(0, nchunk, chunk, 0)
        plsc.subcore_barrier()

        out_base = c * NP + row0
        for j in range(ROWS_PER_TILE // zrows):
            pltpu.sync_copy(acc.at[pl.ds(row0 + j * zrows, zrows)],
                            out_h.at[pl.ds(out_base + j * zrows, zrows)])

    k = pl.kernel(
        body,
        out_type=jax.ShapeDtypeStruct((NC * NP, width), jnp.float32),
        mesh=_MESH,
        compiler_params=pltpu.CompilerParams(use_tc_tiling_on_sc=False),
        scratch_types=[
            pltpu.VMEM_SHARED((NP, width), jnp.float32),
            pltpu.VMEM((CHUNK,), jnp.int32),
            pltpu.VMEM((CHUNK,), jnp.int32),
            pltpu.VMEM((CHUNK, width), jnp.float32),
            pltpu.VMEM((128, width), jnp.float32),
            pltpu.SemaphoreType.DMA,
        ],
    )
    return k(table, srcs_flat, dst)


def _sc_edge_dot(ts, td, ia_flat, ib_flat, ne):
    ept = ne // (NC * NS)
    nchunk = ept // CHUNK
    ngroup = CHUNK // 16
    wdt = 80

    def body(ts_h, td_h, ia_h, ib_h, out_h, idx_a, idx_b, rows_a, rows_b,
             dbuf, sem_a, sem_b):
        c = lax.axis_index("c")
        s = lax.axis_index("s")
        w = c * NS + s
        base = w * ept
        lanes = lax.iota(jnp.int32, 16)

        def chunk(k, _):
            off = base + k * CHUNK
            pltpu.sync_copy(ia_h.at[pl.ds(off, CHUNK)], idx_a)
            pltpu.sync_copy(ib_h.at[pl.ds(off, CHUNK)], idx_b)
            cp_a = pltpu.async_copy(ts_h.at[idx_a], rows_a, sem_a)
            cp_b = pltpu.async_copy(td_h.at[idx_b], rows_b, sem_b)
            cp_a.wait()
            cp_b.wait()
            for g in range(ngroup):
                rows16 = g * 16 + lanes
                accs = [jnp.zeros((16,), jnp.float32) for _ in range(4)]
                for f in range(66):
                    col = jnp.full((16,), f, jnp.int32)
                    a = plsc.load_gather(rows_a, [rows16, col])
                    b = plsc.load_gather(rows_b, [rows16, col])
                    accs[f % 4] = accs[f % 4] + a * b
                dbuf[pl.ds(g * 16, 16)] = (accs[0] + accs[1]) + (accs[2] + accs[3])
            pltpu.sync_copy(dbuf, out_h.at[pl.ds(off, CHUNK)])
            return 0
        lax.fori_loop(0, nchunk, chunk, 0)

    k = pl.kernel(
        body,
        out_type=jax.ShapeDtypeStruct((ne,), jnp.float32),
        mesh=_MESH,
        compiler_params=pltpu.CompilerParams(use_tc_tiling_on_sc=False,
                                             needs_layout_passes=False),
        scratch_types=[
            pltpu.VMEM((CHUNK,), jnp.int32),
            pltpu.VMEM((CHUNK,), jnp.int32),
            pltpu.VMEM((CHUNK, wdt), jnp.float32),
            pltpu.VMEM((CHUNK, wdt), jnp.float32),
            pltpu.VMEM((CHUNK,), jnp.float32),
            pltpu.SemaphoreType.DMA,
            pltpu.SemaphoreType.DMA,
        ],
    )
    return k(ts, td, ia_flat, ib_flat)


def kernel(x, edge_index, batch, num_graphs, W0, b0, W1, b1, Wnm, bnm, Wnl, bnl,
           Wcm, bcm, Wcl, bcl, Wd1, bd1, Wd2, bd2):
    src = edge_index[0]
    dst = edge_index[1]

    p = _sc_agg(x, src, dst, 128, colsplit=False)
    agg1 = p[:N] + p[NP:NP + N]
    h1 = jax.nn.relu((x + agg1) @ W0 + b0)

    p = _sc_agg(h1, src, dst, 64, colsplit=False)
    agg2 = p[:N] + p[NP:NP + N]
    h2 = jax.nn.relu((h1 + agg2) @ W1 + b1)

    emb = jnp.concatenate([h1, h2], axis=1)
    node_mu = emb @ Wnm + bnm
    node_logvar = emb @ Wnl + bnl
    class_mu = emb @ Wcm + bcm
    class_logvar = emb @ Wcl + bcl
    node_kl = -0.5 * jnp.sum(1.0 + node_logvar - node_mu ** 2 - jnp.exp(node_logvar))
    node_kl = 1e-07 * node_kl * num_graphs
    inv_var = jnp.exp(-class_logvar)
    T = jax.ops.segment_sum(inv_var, batch, num_segments=NG)
    Smu = jax.ops.segment_sum(class_mu * inv_var, batch, num_segments=NG)
    gvar = 1.0 / (T + 1e-07)
    gmu = gvar * Smu
    glogvar = jnp.log(gvar + EPS)
    class_kl = -0.5 * jnp.sum(1.0 + glogvar - gmu ** 2 - jnp.exp(glogvar))
    class_kl = 1e-07 * class_kl * num_graphs
    kz = jax.random.key(42)
    eps_n = jax.random.normal(jax.random.fold_in(kz, 0), node_mu.shape, node_mu.dtype)
    z_node = node_mu + eps_n * jnp.exp(0.5 * node_logvar)
    eps_g = jax.random.normal(jax.random.fold_in(kz, 1), gmu.shape, gmu.dtype)
    z_class = gmu[batch] + eps_g[batch] * jnp.exp(0.5 * glogvar)[batch]

    tbl = jnp.concatenate([z_node, z_class], axis=0)
    srcs2 = jnp.concatenate([src, src + N])
    p = _sc_agg(tbl, srcs2, dst, 128, colsplit=True)
    agg3 = jnp.concatenate([p[:N], p[NP:NP + N]], axis=1)

    h = jnp.concatenate([z_node, z_class], axis=1)
    h1d = jax.nn.relu((h + agg3) @ Wd1 + bd1)

    M = Wd2 @ Wd2.T
    v = Wd2 @ bd2
    cc = jnp.dot(bd2, bd2)
    P = h1d @ M
    q = h1d @ v + 0.5 * cc
    one = jnp.ones((N, 1), jnp.float32)
    zpad = jnp.zeros((N, 14), jnp.float32)
    ts = jnp.concatenate([P, q[:, None], one, zpad], axis=1)
    td = jnp.concatenate([h1d, one, q[:, None], zpad], axis=1)

    neg = jax.random.randint(jax.random.fold_in(kz, 2), (2, E), 0, N, dtype=jnp.int32)
    ia = jnp.concatenate([src, neg[0]])
    ib = jnp.concatenate([dst, neg[1]])
    d = _sc_edge_dot(ts, td, ia, ib, 2 * E)
    pos = jax.nn.sigmoid(d[:E])
    pos_loss = -jnp.mean(jnp.log(pos + EPS))
    nv = jax.nn.sigmoid(d[E:])
    neg_loss = -jnp.mean(jnp.log(1.0 - nv + EPS))
    recon_err = pos_loss + neg_loss

    return jnp.stack([recon_err, class_kl, node_kl])


# speedup vs baseline: 5.2680x; 1.6752x over previous
import functools

import jax
import jax.numpy as jnp
from jax import lax
from jax.experimental import pallas as pl
from jax.experimental.pallas import tpu as pltpu
from jax.experimental.pallas import tpu_sc as plsc

N = 10000
E = 320000
NG = 128
EPS = 1e-15

NC = 2
NS = 16
CHUNK = 80
KS = 5
ISLAB = 25
RING = 4
NP = 10240
ROWS_PER_TILE = NP // NS

_MESH = plsc.VectorSubcoreMesh(
    core_axis_name="c", subcore_axis_name="s", num_cores=NC, num_subcores=NS)


def _sc_agg(table, srcs_flat, dst, width, colsplit):
    ept = E // NS if colsplit else E // (NC * NS)
    nchunk = ept // CHUNK
    nslab = nchunk // ISLAB

    def body(table_h, srcs_h, dst_h, zeros_h, out_h, acc, idx_s, idx_d, rows,
             sm0, sm1, sm2, sm3):
        sems = [sm0, sm1, sm2, sm3]
        c = lax.axis_index("c")
        s = lax.axis_index("s")
        row0 = s * ROWS_PER_TILE

        pltpu.sync_copy(zeros_h, acc.at[pl.ds(row0, ROWS_PER_TILE)])
        plsc.subcore_barrier()

        if colsplit:
            src_base = c * E + s * ept
            dst_base = s * ept
        else:
            w = c * NS + s
            src_base = w * ept
            dst_base = w * ept

        def fire(j):
            return pltpu.async_copy(
                table_h.at[idx_s.at[pl.ds(j * CHUNK, CHUNK)]],
                rows.at[pl.ds((j % RING) * CHUNK, CHUNK)], sems[j % RING])

        def slab(m, _):
            off = m * (ISLAB * CHUNK)
            pltpu.sync_copy(srcs_h.at[pl.ds(src_base + off, ISLAB * CHUNK)], idx_s)
            pltpu.sync_copy(dst_h.at[pl.ds(dst_base + off, ISLAB * CHUNK)], idx_d)
            cps = {}
            for j in range(RING - 1):
                cps[j] = fire(j)
            for j in range(ISLAB):
                if j + RING - 1 < ISLAB:
                    cps[j + RING - 1] = fire(j + RING - 1)
                cps[j].wait()
                pltpu.sync_copy(rows.at[pl.ds((j % RING) * CHUNK, CHUNK)],
                                acc.at[idx_d.at[pl.ds(j * CHUNK, CHUNK)]],
                                add=True)
            return 0
        lax.fori_loop(0, nslab, slab, 0)
        plsc.subcore_barrier()

        out_base = c * NP + row0
        for j in range(ROWS_PER_TILE // 128):
            pltpu.sync_copy(acc.at[pl.ds(row0 + j * 128, 128)],
                            out_h.at[pl.ds(out_base + j * 128, 128)])

    k = pl.kernel(
        body,
        out_type=jax.ShapeDtypeStruct((NC * NP, width), jnp.float32),
        mesh=_MESH,
        compiler_params=pltpu.CompilerParams(use_tc_tiling_on_sc=False),
        scratch_types=[
            pltpu.VMEM_SHARED((NP, width), jnp.float32),
            pltpu.VMEM((ISLAB * CHUNK,), jnp.int32),
            pltpu.VMEM((ISLAB * CHUNK,), jnp.int32),
            pltpu.VMEM((RING * CHUNK, width), jnp.float32),
            pltpu.SemaphoreType.DMA,
            pltpu.SemaphoreType.DMA,
            pltpu.SemaphoreType.DMA,
            pltpu.SemaphoreType.DMA,
        ],
    )
    zeros = jnp.zeros((ROWS_PER_TILE, width), jnp.float32)
    return k(table, srcs_flat, dst, zeros)


def _sc_edge_dot(ts, td, ia_flat, ib_flat, ne):
    ept = ne // (NC * NS)
    nchunk = ept // CHUNK
    ngroup = CHUNK // 16
    wdt = 80

    def body(ts_h, td_h, ia_h, ib_h, out_h, idx_a, idx_b, rows_a, rows_b,
             dbuf, *sems):
        c = lax.axis_index("c")
        s = lax.axis_index("s")
        w = c * NS + s
        base = w * ept
        lanes = lax.iota(jnp.int32, 16)

        def slab(m, _):
            off = base + m * (KS * CHUNK)
            pltpu.sync_copy(ia_h.at[pl.ds(off, KS * CHUNK)], idx_a)
            pltpu.sync_copy(ib_h.at[pl.ds(off, KS * CHUNK)], idx_b)
            cps = []
            for j in range(KS):
                sl = pl.ds(j * CHUNK, CHUNK)
                cps.append((
                    pltpu.async_copy(ts_h.at[idx_a.at[sl]], rows_a.at[sl],
                                     sems[2 * j]),
                    pltpu.async_copy(td_h.at[idx_b.at[sl]], rows_b.at[sl],
                                     sems[2 * j + 1])))
            for j in range(KS):
                cps[j][0].wait()
                cps[j][1].wait()
                for g in range(ngroup):
                    rows16 = (j * CHUNK + g * 16) + lanes
                    accs = [jnp.zeros((16,), jnp.float32) for _ in range(4)]
                    for f in range(66):
                        col = jnp.full((16,), f, jnp.int32)
                        a = plsc.load_gather(rows_a, [rows16, col])
                        b = plsc.load_gather(rows_b, [rows16, col])
                        accs[f % 4] = accs[f % 4] + a * b
                    dbuf[pl.ds(j * CHUNK + g * 16, 16)] = (
                        (accs[0] + accs[1]) + (accs[2] + accs[3]))
            pltpu.sync_copy(dbuf, out_h.at[pl.ds(off, KS * CHUNK)])
            return 0
        lax.fori_loop(0, nchunk // KS, slab, 0)

    k = pl.kernel(
        body,
        out_type=jax.ShapeDtypeStruct((ne,), jnp.float32),
        mesh=_MESH,
        compiler_params=pltpu.CompilerParams(use_tc_tiling_on_sc=False,
                                             needs_layout_passes=False),
        scratch_types=[
            pltpu.VMEM((KS * CHUNK,), jnp.int32),
            pltpu.VMEM((KS * CHUNK,), jnp.int32),
            pltpu.VMEM((KS * CHUNK, wdt), jnp.float32),
            pltpu.VMEM((KS * CHUNK, wdt), jnp.float32),
            pltpu.VMEM((KS * CHUNK,), jnp.float32),
        ] + [pltpu.SemaphoreType.DMA] * (2 * KS),
    )
    return k(ts, td, ia_flat, ib_flat)


def kernel(x, edge_index, batch, num_graphs, W0, b0, W1, b1, Wnm, bnm, Wnl, bnl,
           Wcm, bcm, Wcl, bcl, Wd1, bd1, Wd2, bd2):
    src = edge_index[0]
    dst = edge_index[1]

    p = _sc_agg(x, src, dst, 128, colsplit=False)
    agg1 = p[:N] + p[NP:NP + N]
    h1 = jax.nn.relu((x + agg1) @ W0 + b0)

    p = _sc_agg(h1, src, dst, 64, colsplit=False)
    agg2 = p[:N] + p[NP:NP + N]
    h2 = jax.nn.relu((h1 + agg2) @ W1 + b1)

    emb = jnp.concatenate([h1, h2], axis=1)
    node_mu = emb @ Wnm + bnm
    node_logvar = emb @ Wnl + bnl
    class_mu = emb @ Wcm + bcm
    class_logvar = emb @ Wcl + bcl
    node_kl = -0.5 * jnp.sum(1.0 + node_logvar - node_mu ** 2 - jnp.exp(node_logvar))
    node_kl = 1e-07 * node_kl * num_graphs
    inv_var = jnp.exp(-class_logvar)
    T = jax.ops.segment_sum(inv_var, batch, num_segments=NG)
    Smu = jax.ops.segment_sum(class_mu * inv_var, batch, num_segments=NG)
    gvar = 1.0 / (T + 1e-07)
    gmu = gvar * Smu
    glogvar = jnp.log(gvar + EPS)
    class_kl = -0.5 * jnp.sum(1.0 + glogvar - gmu ** 2 - jnp.exp(glogvar))
    class_kl = 1e-07 * class_kl * num_graphs
    kz = jax.random.key(42)
    eps_n = jax.random.normal(jax.random.fold_in(kz, 0), node_mu.shape, node_mu.dtype)
    z_node = node_mu + eps_n * jnp.exp(0.5 * node_logvar)
    eps_g = jax.random.normal(jax.random.fold_in(kz, 1), gmu.shape, gmu.dtype)
    z_class = gmu[batch] + eps_g[batch] * jnp.exp(0.5 * glogvar)[batch]

    tbl = jnp.concatenate([z_node, z_class], axis=0)
    srcs2 = jnp.concatenate([src, src + N])
    p = _sc_agg(tbl, srcs2, dst, 128, colsplit=True)
    agg3 = jnp.concatenate([p[:N], p[NP:NP + N]], axis=1)

    h = jnp.concatenate([z_node, z_class], axis=1)
    h1d = jax.nn.relu((h + agg3) @ Wd1 + bd1)

    M = Wd2 @ Wd2.T
    v = Wd2 @ bd2
    cc = jnp.dot(bd2, bd2)
    P = h1d @ M
    q = h1d @ v + 0.5 * cc
    one = jnp.ones((N, 1), jnp.float32)
    zpad = jnp.zeros((N, 14), jnp.float32)
    ts = jnp.concatenate([P, q[:, None], one, zpad], axis=1)
    td = jnp.concatenate([h1d, one, q[:, None], zpad], axis=1)

    neg = jax.random.randint(jax.random.fold_in(kz, 2), (2, E), 0, N, dtype=jnp.int32)
    ia = jnp.concatenate([src, neg[0]])
    ib = jnp.concatenate([dst, neg[1]])
    d = _sc_edge_dot(ts, td, ia, ib, 2 * E)
    pos = jax.nn.sigmoid(d[:E])
    pos_loss = -jnp.mean(jnp.log(pos + EPS))
    nv = jax.nn.sigmoid(d[E:])
    neg_loss = -jnp.mean(jnp.log(1.0 - nv + EPS))
    recon_err = pos_loss + neg_loss

    return jnp.stack([recon_err, class_kl, node_kl])
